# baseline (speedup 1.0000x reference)
import jax
import jax.numpy as jnp
from jax.experimental import pallas as pl
from jax.experimental.pallas import tpu as pltpu

_ALPHA = 0.25


def _softplus(x):
    return jnp.maximum(x, 0.0) + jnp.log1p(jnp.exp(-jnp.abs(x)))


def _loss_kernel(cls_ref, tcls_ref, pb_ref, pc_ref, gt_ref, gp_ref,
                 mask_ref, idx_ref, out_ref, g_ref, acc_ref, accv_ref):
    i = pl.program_id(0)
    nsteps = pl.num_programs(0)
    B, M, N = g_ref.shape[1], gt_ref.shape[2], g_ref.shape[2]

    @pl.when(i == 0)
    def _box():
        for b in range(B):
            onehot = (jax.lax.broadcasted_iota(jnp.int32, (M, N), 0)
                      == idx_ref[b:b + 1, :]).astype(jnp.float32)
            g_ref[:, b, :] = jnp.dot(gt_ref[b], onehot,
                                     preferred_element_type=jnp.float32)
        g0 = g_ref[0]
        g1 = g_ref[1]
        g2 = g_ref[2]
        g3 = g_ref[3]
        x = gp_ref[0:1, :]
        y = gp_ref[1:2, :]
        l = x - g0
        t = y - g1
        r = g2 - x
        bb = g3 - y
        ctr = jnp.sqrt(
            jnp.minimum(l, r) / jnp.maximum(jnp.maximum(l, r), 1e-6)
            * (jnp.minimum(t, bb) / jnp.maximum(jnp.maximum(t, bb), 1e-6)))
        mask = mask_ref[...]
        pc = pc_ref[:, 0, :]
        center_sum = jnp.sum((_softplus(pc) - pc * ctr) * mask)

        pred_cx = x + pb_ref[:, 0, :]
        pred_cy = y + pb_ref[:, 1, :]
        w = _softplus(jnp.exp(pb_ref[:, 2, :]))
        h = _softplus(jnp.exp(pb_ref[:, 3, :]))
        px0 = pred_cx - 0.5 * w
        py0 = pred_cy - 0.5 * h
        px1 = pred_cx + 0.5 * w
        py1 = pred_cy + 0.5 * h
        ixmin = jnp.maximum(px0, g0)
        iymin = jnp.maximum(py0, g1)
        ixmax = jnp.minimum(px1, g2)
        iymax = jnp.minimum(py1, g3)
        inter = (jnp.maximum(ixmax - ixmin, 0.0)
                 * jnp.maximum(iymax - iymin, 0.0))
        pred_area = (px1 - px0) * (py1 - py0)
        gt_area = (g2 - g0) * (g3 - g1)
        union = pred_area + gt_area - inter + 1e-6
        iou = inter / (union + 1e-6)
        exmin = jnp.minimum(px0, g0)
        eymin = jnp.minimum(py0, g1)
        exmax = jnp.maximum(px1, g2)
        eymax = jnp.maximum(py1, g3)
        enclose = (jnp.maximum(exmax - exmin, 0.0)
                   * jnp.maximum(eymax - eymin, 0.0) + 1e-6)
        giou = iou - (enclose - union) / (enclose + 1e-6)
        reg_sum = jnp.sum((1.0 - giou) * ctr * mask)

        acc_ref[1] = jnp.sum(mask)
        acc_ref[2] = reg_sum
        acc_ref[3] = center_sum
        accv_ref[...] = jnp.zeros_like(accv_ref)

    xl = cls_ref[...]
    tl = tcls_ref[...]
    bce = jnp.log1p(jnp.exp(xl)) - xl * tl
    one_m_pt = 1.0 - jnp.exp(-bce)
    focal = one_m_pt * one_m_pt * bce
    ones = jnp.ones((1, focal.shape[0]), dtype=jnp.float32)
    accv_ref[...] += jnp.dot(ones, focal, preferred_element_type=jnp.float32)

    @pl.when(i == nsteps - 1)
    def _finalize():
        denom = jnp.maximum(acc_ref[1], 1.0)
        loss_cls = _ALPHA * jnp.sum(accv_ref[...]) / denom
        loss_reg = acc_ref[2] / denom
        loss_center = acc_ref[3] / denom
        out_ref[0] = loss_cls + 2.0 * loss_reg + loss_center
        out_ref[1] = loss_cls
        out_ref[2] = loss_reg
        out_ref[3] = loss_center


def kernel(pred_bboxes, pred_cls, pred_center, gt_boxes_xyxy, grid_points,
           target_cls, is_positive, matched_obj_idx):
    B, C, N = pred_cls.shape
    M = gt_boxes_xyxy.shape[1]
    STEPS = 16
    RC = (B * C) // STEPS

    cls2 = pred_cls.reshape(B * C, N)
    tcls2 = target_cls.reshape(B * C, N)
    gt_t = jnp.swapaxes(gt_boxes_xyxy, 1, 2)
    gp_t = grid_points.T
    mask = is_positive.astype(jnp.float32)
    idx = matched_obj_idx.astype(jnp.int32)

    out = pl.pallas_call(
        _loss_kernel,
        grid=(STEPS,),
        in_specs=[
            pl.BlockSpec((RC, N), lambda i: (i, 0)),
            pl.BlockSpec((RC, N), lambda i: (i, 0)),
            pl.BlockSpec((B, 4, N), lambda i: (0, 0, 0)),
            pl.BlockSpec((B, 1, N), lambda i: (0, 0, 0)),
            pl.BlockSpec((B, 4, M), lambda i: (0, 0, 0)),
            pl.BlockSpec((2, N), lambda i: (0, 0)),
            pl.BlockSpec((B, N), lambda i: (0, 0)),
            pl.BlockSpec((B, N), lambda i: (0, 0)),
        ],
        out_specs=pl.BlockSpec(memory_space=pltpu.SMEM,
                               block_shape=(4,), index_map=lambda i: (0,)),
        out_shape=jax.ShapeDtypeStruct((4,), jnp.float32),
        scratch_shapes=[
            pltpu.VMEM((4, B, N), jnp.float32),
            pltpu.SMEM((4,), jnp.float32),
            pltpu.VMEM((1, N), jnp.float32),
        ],
        compiler_params=pltpu.CompilerParams(
            dimension_semantics=("arbitrary",)),
    )(cls2, tcls2, pred_bboxes, pred_center, gt_t, gp_t, mask, idx)
    return (out[0], out[1], out[2], out[3])


# speedup vs baseline: 1.0336x; 1.0336x over previous
import jax
import jax.numpy as jnp
from jax.experimental import pallas as pl
from jax.experimental.pallas import tpu as pltpu

_ALPHA = 0.25


def _softplus(x):
    return jnp.maximum(x, 0.0) + jnp.log1p(jnp.exp(-jnp.abs(x)))


def _loss_kernel(cls_ref, tcls_ref, pb_ref, pc_ref, gt_ref, gp_ref,
                 mask_ref, idx_ref, out_ref, g_ref, acc_ref, accv_ref):
    i = pl.program_id(0)
    nsteps = pl.num_programs(0)
    B, M, N = g_ref.shape[1], gt_ref.shape[2], g_ref.shape[2]

    @pl.when(i == 0)
    def _box():
        for b in range(B):
            onehot = (jax.lax.broadcasted_iota(jnp.int32, (M, N), 0)
                      == idx_ref[b:b + 1, :]).astype(jnp.float32)
            g_ref[:, b, :] = jnp.dot(gt_ref[b], onehot,
                                     preferred_element_type=jnp.float32)
        g0 = g_ref[0]
        g1 = g_ref[1]
        g2 = g_ref[2]
        g3 = g_ref[3]
        x = gp_ref[0:1, :]
        y = gp_ref[1:2, :]
        l = x - g0
        t = y - g1
        r = g2 - x
        bb = g3 - y
        ctr = jnp.sqrt(
            jnp.minimum(l, r) / jnp.maximum(jnp.maximum(l, r), 1e-6)
            * (jnp.minimum(t, bb) / jnp.maximum(jnp.maximum(t, bb), 1e-6)))
        mask = mask_ref[...]
        pc = pc_ref[:, 0, :]
        center_sum = jnp.sum((_softplus(pc) - pc * ctr) * mask)

        pred_cx = x + pb_ref[:, 0, :]
        pred_cy = y + pb_ref[:, 1, :]
        w = _softplus(jnp.exp(pb_ref[:, 2, :]))
        h = _softplus(jnp.exp(pb_ref[:, 3, :]))
        px0 = pred_cx - 0.5 * w
        py0 = pred_cy - 0.5 * h
        px1 = pred_cx + 0.5 * w
        py1 = pred_cy + 0.5 * h
        ixmin = jnp.maximum(px0, g0)
        iymin = jnp.maximum(py0, g1)
        ixmax = jnp.minimum(px1, g2)
        iymax = jnp.minimum(py1, g3)
        inter = (jnp.maximum(ixmax - ixmin, 0.0)
                 * jnp.maximum(iymax - iymin, 0.0))
        pred_area = (px1 - px0) * (py1 - py0)
        gt_area = (g2 - g0) * (g3 - g1)
        union = pred_area + gt_area - inter + 1e-6
        iou = inter / (union + 1e-6)
        exmin = jnp.minimum(px0, g0)
        eymin = jnp.minimum(py0, g1)
        exmax = jnp.maximum(px1, g2)
        eymax = jnp.maximum(py1, g3)
        enclose = (jnp.maximum(exmax - exmin, 0.0)
                   * jnp.maximum(eymax - eymin, 0.0) + 1e-6)
        giou = iou - (enclose - union) / (enclose + 1e-6)
        reg_sum = jnp.sum((1.0 - giou) * ctr * mask)

        acc_ref[1] = jnp.sum(mask)
        acc_ref[2] = reg_sum
        acc_ref[3] = center_sum
        accv_ref[...] = jnp.zeros_like(accv_ref)

    xl = cls_ref[...]
    tl = tcls_ref[...]
    bce = jnp.log1p(jnp.exp(xl)) - xl * tl
    one_m_pt = 1.0 - jnp.exp(-bce)
    focal = one_m_pt * one_m_pt * bce
    ones = jnp.ones((1, focal.shape[0]), dtype=jnp.float32)
    accv_ref[...] += jnp.dot(ones, focal, preferred_element_type=jnp.float32)

    @pl.when(i == nsteps - 1)
    def _finalize():
        denom = jnp.maximum(acc_ref[1], 1.0)
        loss_cls = _ALPHA * jnp.sum(accv_ref[...]) / denom
        loss_reg = acc_ref[2] / denom
        loss_center = acc_ref[3] / denom
        out_ref[0] = loss_cls + 2.0 * loss_reg + loss_center
        out_ref[1] = loss_cls
        out_ref[2] = loss_reg
        out_ref[3] = loss_center


def kernel(pred_bboxes, pred_cls, pred_center, gt_boxes_xyxy, grid_points,
           target_cls, is_positive, matched_obj_idx):
    B, C, N = pred_cls.shape
    M = gt_boxes_xyxy.shape[1]
    STEPS = 4
    RC = (B * C) // STEPS

    cls2 = pred_cls.reshape(B * C, N)
    tcls2 = target_cls.reshape(B * C, N)
    gt_t = jnp.swapaxes(gt_boxes_xyxy, 1, 2)
    gp_t = grid_points.T
    mask = is_positive.astype(jnp.float32)
    idx = matched_obj_idx.astype(jnp.int32)

    out = pl.pallas_call(
        _loss_kernel,
        grid=(STEPS,),
        in_specs=[
            pl.BlockSpec((RC, N), lambda i: (i, 0)),
            pl.BlockSpec((RC, N), lambda i: (i, 0)),
            pl.BlockSpec((B, 4, N), lambda i: (0, 0, 0)),
            pl.BlockSpec((B, 1, N), lambda i: (0, 0, 0)),
            pl.BlockSpec((B, 4, M), lambda i: (0, 0, 0)),
            pl.BlockSpec((2, N), lambda i: (0, 0)),
            pl.BlockSpec((B, N), lambda i: (0, 0)),
            pl.BlockSpec((B, N), lambda i: (0, 0)),
        ],
        out_specs=pl.BlockSpec(memory_space=pltpu.SMEM,
                               block_shape=(4,), index_map=lambda i: (0,)),
        out_shape=jax.ShapeDtypeStruct((4,), jnp.float32),
        scratch_shapes=[
            pltpu.VMEM((4, B, N), jnp.float32),
            pltpu.SMEM((4,), jnp.float32),
            pltpu.VMEM((1, N), jnp.float32),
        ],
        compiler_params=pltpu.CompilerParams(
            dimension_semantics=("arbitrary",)),
    )(cls2, tcls2, pred_bboxes, pred_center, gt_t, gp_t, mask, idx)
    return (out[0], out[1], out[2], out[3])


# speedup vs baseline: 1.0861x; 1.0508x over previous
import jax
import jax.numpy as jnp
from jax.experimental import pallas as pl
from jax.experimental.pallas import tpu as pltpu

_ALPHA = 0.25


def _softplus(x):
    return jnp.maximum(x, 0.0) + jnp.log1p(jnp.exp(-jnp.abs(x)))


def _loss_kernel(cls_ref, tcls_ref, pb_ref, pc_ref, gt_ref, gp_ref,
                 mask_ref, idx_ref, out_ref, g_ref, acc_ref, accv_ref):
    i = pl.program_id(0)
    nsteps = pl.num_programs(0)
    B, M, N = g_ref.shape[1], gt_ref.shape[2], g_ref.shape[2]

    @pl.when(i == 0)
    def _box():
        for b in range(B):
            onehot = (jax.lax.broadcasted_iota(jnp.int32, (M, N), 0)
                      == idx_ref[b:b + 1, :]).astype(jnp.float32)
            g_ref[:, b, :] = jnp.dot(gt_ref[b], onehot,
                                     preferred_element_type=jnp.float32)
        g0 = g_ref[0]
        g1 = g_ref[1]
        g2 = g_ref[2]
        g3 = g_ref[3]
        x = gp_ref[0:1, :]
        y = gp_ref[1:2, :]
        l = x - g0
        t = y - g1
        r = g2 - x
        bb = g3 - y
        ctr = jnp.sqrt(
            jnp.minimum(l, r) / jnp.maximum(jnp.maximum(l, r), 1e-6)
            * (jnp.minimum(t, bb) / jnp.maximum(jnp.maximum(t, bb), 1e-6)))
        mask = mask_ref[...]
        pc = pc_ref[:, 0, :]
        center_sum = jnp.sum((_softplus(pc) - pc * ctr) * mask)

        pred_cx = x + pb_ref[:, 0, :]
        pred_cy = y + pb_ref[:, 1, :]
        w = _softplus(jnp.exp(pb_ref[:, 2, :]))
        h = _softplus(jnp.exp(pb_ref[:, 3, :]))
        px0 = pred_cx - 0.5 * w
        py0 = pred_cy - 0.5 * h
        px1 = pred_cx + 0.5 * w
        py1 = pred_cy + 0.5 * h
        ixmin = jnp.maximum(px0, g0)
        iymin = jnp.maximum(py0, g1)
        ixmax = jnp.minimum(px1, g2)
        iymax = jnp.minimum(py1, g3)
        inter = (jnp.maximum(ixmax - ixmin, 0.0)
                 * jnp.maximum(iymax - iymin, 0.0))
        pred_area = (px1 - px0) * (py1 - py0)
        gt_area = (g2 - g0) * (g3 - g1)
        union = pred_area + gt_area - inter + 1e-6
        iou = inter / (union + 1e-6)
        exmin = jnp.minimum(px0, g0)
        eymin = jnp.minimum(py0, g1)
        exmax = jnp.maximum(px1, g2)
        eymax = jnp.maximum(py1, g3)
        enclose = (jnp.maximum(exmax - exmin, 0.0)
                   * jnp.maximum(eymax - eymin, 0.0) + 1e-6)
        giou = iou - (enclose - union) / (enclose + 1e-6)
        reg_sum = jnp.sum((1.0 - giou) * ctr * mask)

        acc_ref[1] = jnp.sum(mask)
        acc_ref[2] = reg_sum
        acc_ref[3] = center_sum
        accv_ref[...] = jnp.zeros_like(accv_ref)

    xl = cls_ref[...]
    tl = tcls_ref[...]
    bce = jnp.log1p(jnp.exp(xl)) - xl * tl
    one_m_pt = 1.0 - jnp.exp2(bce * -1.4426950408889634)
    focal = one_m_pt * one_m_pt * bce
    ones = jnp.ones((1, focal.shape[0]), dtype=jnp.float32)
    accv_ref[...] += jnp.dot(ones, focal, preferred_element_type=jnp.float32)

    @pl.when(i == nsteps - 1)
    def _finalize():
        denom = jnp.maximum(acc_ref[1], 1.0)
        loss_cls = _ALPHA * jnp.sum(accv_ref[...]) / denom
        loss_reg = acc_ref[2] / denom
        loss_center = acc_ref[3] / denom
        out_ref[0] = loss_cls + 2.0 * loss_reg + loss_center
        out_ref[1] = loss_cls
        out_ref[2] = loss_reg
        out_ref[3] = loss_center


def kernel(pred_bboxes, pred_cls, pred_center, gt_boxes_xyxy, grid_points,
           target_cls, is_positive, matched_obj_idx):
    B, C, N = pred_cls.shape
    M = gt_boxes_xyxy.shape[1]
    STEPS = 8
    RC = (B * C) // STEPS

    cls2 = pred_cls.reshape(B * C, N)
    tcls2 = target_cls.reshape(B * C, N)
    gt_t = jnp.swapaxes(gt_boxes_xyxy, 1, 2)
    gp_t = grid_points.T
    mask = is_positive.astype(jnp.float32)
    idx = matched_obj_idx.astype(jnp.int32)

    out = pl.pallas_call(
        _loss_kernel,
        grid=(STEPS,),
        in_specs=[
            pl.BlockSpec((RC, N), lambda i: (i, 0)),
            pl.BlockSpec((RC, N), lambda i: (i, 0)),
            pl.BlockSpec((B, 4, N), lambda i: (0, 0, 0)),
            pl.BlockSpec((B, 1, N), lambda i: (0, 0, 0)),
            pl.BlockSpec((B, 4, M), lambda i: (0, 0, 0)),
            pl.BlockSpec((2, N), lambda i: (0, 0)),
            pl.BlockSpec((B, N), lambda i: (0, 0)),
            pl.BlockSpec((B, N), lambda i: (0, 0)),
        ],
        out_specs=pl.BlockSpec(memory_space=pltpu.SMEM,
                               block_shape=(4,), index_map=lambda i: (0,)),
        out_shape=jax.ShapeDtypeStruct((4,), jnp.float32),
        scratch_shapes=[
            pltpu.VMEM((4, B, N), jnp.float32),
            pltpu.SMEM((4,), jnp.float32),
            pltpu.VMEM((1, N), jnp.float32),
        ],
        compiler_params=pltpu.CompilerParams(
            dimension_semantics=("arbitrary",)),
    )(cls2, tcls2, pred_bboxes, pred_center, gt_t, gp_t, mask, idx)
    return (out[0], out[1], out[2], out[3])


# speedup vs baseline: 1.0899x; 1.0035x over previous
import jax
import jax.numpy as jnp
from jax.experimental import pallas as pl
from jax.experimental.pallas import tpu as pltpu

_ALPHA = 0.25


def _softplus(x):
    return jnp.maximum(x, 0.0) + jnp.log1p(jnp.exp(-jnp.abs(x)))


def _loss_kernel(cls_ref, tcls_ref, pb_ref, pc_ref, gt_ref, gp_ref,
                 mask_ref, idx_ref, out_ref, g_ref, acc_ref, accv_ref):
    i = pl.program_id(0)
    nsteps = pl.num_programs(0)
    B, M, N = g_ref.shape[1], gt_ref.shape[2], g_ref.shape[2]

    @pl.when(i == 0)
    def _box():
        for b in range(B):
            onehot = jnp.where(
                jax.lax.broadcasted_iota(jnp.int32, (M, N), 0)
                == idx_ref[b:b + 1, :], 1.0, 0.0)
            g_ref[:, b, :] = jnp.dot(gt_ref[b], onehot,
                                     preferred_element_type=jnp.float32)
        g0 = g_ref[0]
        g1 = g_ref[1]
        g2 = g_ref[2]
        g3 = g_ref[3]
        x = gp_ref[0:1, :]
        y = gp_ref[1:2, :]
        l = x - g0
        t = y - g1
        r = g2 - x
        bb = g3 - y
        ctr = jnp.sqrt(
            jnp.minimum(l, r) / jnp.maximum(jnp.maximum(l, r), 1e-6)
            * (jnp.minimum(t, bb) / jnp.maximum(jnp.maximum(t, bb), 1e-6)))
        mask = mask_ref[...]
        pc = pc_ref[:, 0, :]
        center_sum = jnp.sum((_softplus(pc) - pc * ctr) * mask)

        pred_cx = x + pb_ref[:, 0, :]
        pred_cy = y + pb_ref[:, 1, :]
        w = _softplus(jnp.exp(pb_ref[:, 2, :]))
        h = _softplus(jnp.exp(pb_ref[:, 3, :]))
        px0 = pred_cx - 0.5 * w
        py0 = pred_cy - 0.5 * h
        px1 = pred_cx + 0.5 * w
        py1 = pred_cy + 0.5 * h
        ixmin = jnp.maximum(px0, g0)
        iymin = jnp.maximum(py0, g1)
        ixmax = jnp.minimum(px1, g2)
        iymax = jnp.minimum(py1, g3)
        inter = (jnp.maximum(ixmax - ixmin, 0.0)
                 * jnp.maximum(iymax - iymin, 0.0))
        pred_area = (px1 - px0) * (py1 - py0)
        gt_area = (g2 - g0) * (g3 - g1)
        union = pred_area + gt_area - inter + 1e-6
        iou = inter / (union + 1e-6)
        exmin = jnp.minimum(px0, g0)
        eymin = jnp.minimum(py0, g1)
        exmax = jnp.maximum(px1, g2)
        eymax = jnp.maximum(py1, g3)
        enclose = (jnp.maximum(exmax - exmin, 0.0)
                   * jnp.maximum(eymax - eymin, 0.0) + 1e-6)
        giou = iou - (enclose - union) / (enclose + 1e-6)
        reg_sum = jnp.sum((1.0 - giou) * ctr * mask)

        acc_ref[1] = jnp.sum(mask)
        acc_ref[2] = reg_sum
        acc_ref[3] = center_sum
        accv_ref[...] = jnp.zeros_like(accv_ref)

    xl = cls_ref[...]
    tl = tcls_ref[...]
    bce = jnp.log1p(jnp.exp(xl)) - xl * tl
    one_m_pt = 1.0 - jnp.exp2(bce * -1.4426950408889634)
    focal = one_m_pt * one_m_pt * bce
    ones = jnp.ones((1, focal.shape[0]), dtype=jnp.float32)
    accv_ref[...] += jnp.dot(ones, focal, preferred_element_type=jnp.float32)

    @pl.when(i == nsteps - 1)
    def _finalize():
        denom = jnp.maximum(acc_ref[1], 1.0)
        loss_cls = _ALPHA * jnp.sum(accv_ref[...]) / denom
        loss_reg = acc_ref[2] / denom
        loss_center = acc_ref[3] / denom
        out_ref[0] = loss_cls + 2.0 * loss_reg + loss_center
        out_ref[1] = loss_cls
        out_ref[2] = loss_reg
        out_ref[3] = loss_center


def kernel(pred_bboxes, pred_cls, pred_center, gt_boxes_xyxy, grid_points,
           target_cls, is_positive, matched_obj_idx):
    B, C, N = pred_cls.shape
    M = gt_boxes_xyxy.shape[1]
    STEPS = 8
    RC = (B * C) // STEPS

    cls2 = pred_cls.reshape(B * C, N)
    tcls2 = target_cls.reshape(B * C, N)
    gt_t = jnp.swapaxes(gt_boxes_xyxy, 1, 2)
    gp_t = grid_points.T
    mask = is_positive.astype(jnp.float32)
    idx = matched_obj_idx.astype(jnp.int32)

    out = pl.pallas_call(
        _loss_kernel,
        grid=(STEPS,),
        in_specs=[
            pl.BlockSpec((RC, N), lambda i: (i, 0)),
            pl.BlockSpec((RC, N), lambda i: (i, 0)),
            pl.BlockSpec((B, 4, N), lambda i: (0, 0, 0)),
            pl.BlockSpec((B, 1, N), lambda i: (0, 0, 0)),
            pl.BlockSpec((B, 4, M), lambda i: (0, 0, 0)),
            pl.BlockSpec((2, N), lambda i: (0, 0)),
            pl.BlockSpec((B, N), lambda i: (0, 0)),
            pl.BlockSpec((B, N), lambda i: (0, 0)),
        ],
        out_specs=pl.BlockSpec(memory_space=pltpu.SMEM,
                               block_shape=(4,), index_map=lambda i: (0,)),
        out_shape=jax.ShapeDtypeStruct((4,), jnp.float32),
        scratch_shapes=[
            pltpu.VMEM((4, B, N), jnp.float32),
            pltpu.SMEM((4,), jnp.float32),
            pltpu.VMEM((1, N), jnp.float32),
        ],
        compiler_params=pltpu.CompilerParams(
            dimension_semantics=("arbitrary",)),
    )(cls2, tcls2, pred_bboxes, pred_center, gt_t, gp_t, mask, idx)
    return (out[0], out[1], out[2], out[3])
